# baseline (speedup 1.0000x reference)
import functools

import numpy as np

import jax
import jax.numpy as jnp
from jax import lax
from jax.experimental import pallas as pl
from jax.experimental.pallas import tpu as pltpu
from jax.experimental.pallas import tpu_sc as plsc

L = 16


def _pos_encoding_table_np(seq_len, d_model):
    pos = np.arange(seq_len, dtype=np.float64)[:, None]
    _2i = np.arange(0, d_model, 2, dtype=np.float64)
    angle = pos / np.power(10000.0, _2i / d_model)
    pe = np.empty((seq_len, d_model), dtype=np.float32)
    pe[:, 0::2] = np.sin(angle.astype(np.float32))
    pe[:, 1::2] = np.cos(angle.astype(np.float32))
    return pe


def _rsqrt16(v):
    half = v * 0.5
    i = plsc.bitcast(v, jnp.int32)
    i = 0x5F3759DF - (i >> 1)
    r = plsc.bitcast(i, jnp.float32)
    r = r * (1.5 - half * r * r)
    r = r * (1.5 - half * r * r)
    r = r * (1.5 - half * r * r)
    return r


def _make_sc_kernel(B, S, D, V, SB):
    info = plsc.get_sparse_core_info()
    NC, NS = info.num_cores, info.num_subcores
    NW = NC * NS
    assert S % NW == 0
    s_per_w = S // NW
    assert s_per_w % SB == 0 and SB == L
    n_sb = s_per_w // SB
    n_slices = D // L

    mesh = plsc.VectorSubcoreMesh(core_axis_name="c", subcore_axis_name="s")

    @functools.partial(
        pl.kernel,
        mesh=mesh,
        out_type=jax.ShapeDtypeStruct((B, S, D), jnp.float32),
        compiler_params=pltpu.CompilerParams(needs_layout_passes=False),
        scratch_types=[
            pltpu.VMEM((SB,), jnp.int32),
            pltpu.VMEM((SB, D), jnp.float32),
            pltpu.VMEM((2 * SB, D), jnp.float32),
            pltpu.VMEM((D,), jnp.float32),
            pltpu.VMEM((D,), jnp.float32),
            pltpu.SemaphoreType.DMA,
            pltpu.SemaphoreType.DMA,
            pltpu.SemaphoreType.DMA,
        ],
    )
    def k(table_hbm, ids_hbm, pe_hbm, gamma_hbm, beta_hbm, out_hbm,
          idx_v, pe_v, rows_all, gamma_v, beta_v, sem, sw0, sw1):
        wid = lax.axis_index("s") * NC + lax.axis_index("c")
        s0 = wid * s_per_w

        pltpu.sync_copy(gamma_hbm, gamma_v)
        pltpu.sync_copy(beta_hbm, beta_v)

        inv_d = jnp.float32(1.0 / D)
        lanes = lax.iota(jnp.int32, L)
        zf = jnp.zeros((L,), jnp.float32)

        def wo_copy(g, kk):
            b = g % B
            s_base = s0 + (g // B) * SB
            return pltpu.make_async_copy(
                rows_all.at[pl.ds(kk * SB, SB)],
                out_hbm.at[b, pl.ds(s_base, SB)],
                (sw0, sw1)[kk])

        def start_writeout(g):
            for kk in range(2):
                @pl.when(g % 2 == kk)
                def _():
                    wo_copy(g, kk).start()

        def wait_writeout(g):
            for kk in range(2):
                @pl.when(g % 2 == kk)
                def _():
                    wo_copy(g, kk).wait()

        def chunk_body(g, _):
            sb = g // B
            b = g % B
            s_base = s0 + sb * SB
            base = (g % 2) * SB
            rows_v = rows_all.at[pl.ds(base, SB)]

            @pl.when(b == 0)
            def _():
                pltpu.sync_copy(pe_hbm.at[pl.ds(s_base, SB)], pe_v)

            @pl.when(g >= 2)
            def _():
                wait_writeout(g - 2)

            pltpu.sync_copy(ids_hbm.at[b, pl.ds(s_base, SB)], idx_v)
            pltpu.async_copy(table_hbm.at[idx_v], rows_v, sem).wait()

            def t_body(t, carry):
                sum_vec, ssq_vec = carry
                a = [zf, zf]
                q = [zf, zf]
                bt = base + t
                for j in range(n_slices):
                    sl = pl.ds(j * L, L)
                    y = rows_all[bt, sl] + pe_v[t, sl]
                    rows_all[bt, sl] = y
                    a[j % 2] = a[j % 2] + y
                    q[j % 2] = q[j % 2] + y * y
                s = jnp.sum(a[0] + a[1])
                ss = jnp.sum(q[0] + q[1])
                lane = lanes == t
                return (jnp.where(lane, s, sum_vec),
                        jnp.where(lane, ss, ssq_vec))

            sum_vec, ssq_vec = lax.fori_loop(0, SB, t_body, (zf, zf))
            mean_vec = sum_vec * inv_d
            var_vec = ssq_vec * inv_d - mean_vec * mean_vec
            rinv_vec = _rsqrt16(var_vec + jnp.float32(1e-5))

            msp = [jnp.full((L,), mean_vec[t], jnp.float32) for t in range(SB)]
            rsp = [jnp.full((L,), rinv_vec[t], jnp.float32) for t in range(SB)]

            def j_body(j, _):
                sl = pl.ds(j * L, L)
                gj = gamma_v[sl]
                bj = beta_v[sl]
                for t in range(SB):
                    y = rows_all[base + t, sl]
                    rows_all[base + t, sl] = (y - msp[t]) * rsp[t] * gj + bj
                return 0

            lax.fori_loop(0, n_slices, j_body, 0)

            start_writeout(g)
            return 0

        n_ch = n_sb * B
        lax.fori_loop(0, n_ch, chunk_body, 0)
        wait_writeout(n_ch - 2)
        wait_writeout(n_ch - 1)

    return k


def kernel(trg_ids, emb_table, gamma, beta):
    B, S = trg_ids.shape
    V, D = emb_table.shape
    pe = jnp.asarray(_pos_encoding_table_np(S, D))
    k = _make_sc_kernel(B, S, D, V, SB=16)
    return k(emb_table, trg_ids.astype(jnp.int32), pe, gamma, beta)


# speedup vs baseline: 1.5691x; 1.5691x over previous
import functools

import numpy as np

import jax
import jax.numpy as jnp
from jax import lax
from jax.experimental import pallas as pl
from jax.experimental.pallas import tpu as pltpu
from jax.experimental.pallas import tpu_sc as plsc

L = 16


def _pos_encoding_table_np(seq_len, d_model):
    pos = np.arange(seq_len, dtype=np.float64)[:, None]
    _2i = np.arange(0, d_model, 2, dtype=np.float64)
    angle = pos / np.power(10000.0, _2i / d_model)
    pe = np.empty((seq_len, d_model), dtype=np.float32)
    pe[:, 0::2] = np.sin(angle.astype(np.float32))
    pe[:, 1::2] = np.cos(angle.astype(np.float32))
    return pe


def _rsqrt16(v):
    half = v * 0.5
    i = plsc.bitcast(v, jnp.int32)
    i = 0x5F3759DF - (i >> 1)
    r = plsc.bitcast(i, jnp.float32)
    r = r * (1.5 - half * r * r)
    r = r * (1.5 - half * r * r)
    r = r * (1.5 - half * r * r)
    return r


def _make_sc_kernel(B, S, D, V, SB):
    info = plsc.get_sparse_core_info()
    NC, NS = info.num_cores, info.num_subcores
    NW = NC * NS
    assert S % NW == 0
    s_per_w = S // NW
    assert s_per_w % SB == 0 and SB == L
    n_sb = s_per_w // SB
    n_slices = D // L
    n_ch = n_sb * B
    assert n_ch % 2 == 0 and B % 2 == 0
    n_pairs = n_ch // 2

    mesh = plsc.VectorSubcoreMesh(core_axis_name="c", subcore_axis_name="s")

    @functools.partial(
        pl.kernel,
        mesh=mesh,
        out_type=jax.ShapeDtypeStruct((B, S, D), jnp.float32),
        compiler_params=pltpu.CompilerParams(needs_layout_passes=False),
        scratch_types=[
            pltpu.VMEM((SB,), jnp.int32),
            pltpu.VMEM((SB,), jnp.int32),
            pltpu.VMEM((SB, D), jnp.float32),
            pltpu.VMEM((SB, D), jnp.float32),
            pltpu.VMEM((SB, D), jnp.float32),
            pltpu.VMEM((D,), jnp.float32),
            pltpu.VMEM((D,), jnp.float32),
            pltpu.SemaphoreType.DMA,
            pltpu.SemaphoreType.DMA,
            pltpu.SemaphoreType.DMA,
            pltpu.SemaphoreType.DMA,
        ],
    )
    def k(table_hbm, ids_hbm, pe_hbm, gamma_hbm, beta_hbm, out_hbm,
          idx0, idx1, pe_v, rows0, rows1, gamma_v, beta_v,
          sg0, sg1, sw0, sw1):
        wid = lax.axis_index("s") * NC + lax.axis_index("c")
        s0 = wid * s_per_w

        pltpu.sync_copy(gamma_hbm, gamma_v)
        pltpu.sync_copy(beta_hbm, beta_v)

        inv_d = jnp.float32(1.0 / D)
        lanes = lax.iota(jnp.int32, L)
        zf = jnp.zeros((L,), jnp.float32)
        idxs = (idx0, idx1)
        rows = (rows0, rows1)
        sgs = (sg0, sg1)
        sws = (sw0, sw1)

        def start_gather(kk, b, s_base):
            pltpu.sync_copy(ids_hbm.at[b, pl.ds(s_base, SB)], idxs[kk])
            pltpu.make_async_copy(table_hbm.at[idxs[kk]], rows[kk],
                                  sgs[kk]).start()

        def wait_gather(kk):
            pltpu.make_async_copy(table_hbm.at[idxs[kk]], rows[kk],
                                  sgs[kk]).wait()

        def start_writeout(kk, b, s_base):
            pltpu.make_async_copy(rows[kk], out_hbm.at[b, pl.ds(s_base, SB)],
                                  sws[kk]).start()

        def wait_writeout(kk, b, s_base):
            pltpu.make_async_copy(rows[kk], out_hbm.at[b, pl.ds(s_base, SB)],
                                  sws[kk]).wait()

        def process(rows_v):
            def t_body(t, carry):
                sum_vec, ssq_vec = carry
                a = [zf, zf, zf, zf]
                q = [zf, zf, zf, zf]
                for j in range(n_slices):
                    sl = pl.ds(j * L, L)
                    y = rows_v[t, sl] + pe_v[t, sl]
                    rows_v[t, sl] = y
                    a[j % 4] = a[j % 4] + y
                    q[j % 4] = q[j % 4] + y * y
                s = jnp.sum((a[0] + a[1]) + (a[2] + a[3]))
                ss = jnp.sum((q[0] + q[1]) + (q[2] + q[3]))
                lane = lanes == t
                return (jnp.where(lane, s, sum_vec),
                        jnp.where(lane, ss, ssq_vec))

            sum_vec, ssq_vec = lax.fori_loop(0, SB, t_body, (zf, zf))
            mean_vec = sum_vec * inv_d
            var_vec = ssq_vec * inv_d - mean_vec * mean_vec
            rinv_vec = _rsqrt16(var_vec + jnp.float32(1e-5))

            msp = [jnp.full((L,), mean_vec[t], jnp.float32) for t in range(SB)]
            rsp = [jnp.full((L,), rinv_vec[t], jnp.float32) for t in range(SB)]

            def j_body(j, _):
                sl = pl.ds(j * L, L)
                gj = gamma_v[sl]
                bj = beta_v[sl]
                for t in range(SB):
                    y = rows_v[t, sl]
                    rows_v[t, sl] = (y - msp[t]) * rsp[t] * gj + bj
                return 0

            lax.fori_loop(0, n_slices, j_body, 0)

        start_gather(0, 0, s0)

        def pair_body(p, _):
            g0 = p * 2
            b0 = g0 % B
            b1 = b0 + 1
            s_base = s0 + (g0 // B) * SB

            wait_gather(0)

            @pl.when(b0 == 0)
            def _():
                pltpu.sync_copy(pe_hbm.at[pl.ds(s_base, SB)], pe_v)

            @pl.when(p > 0)
            def _():
                wait_writeout(1, b1, s_base)
            start_gather(1, b1, s_base)

            process(rows0)
            start_writeout(0, b0, s_base)

            wait_gather(1)

            @pl.when(p + 1 < n_pairs)
            def _():
                g_n = g0 + 2
                wait_writeout(0, b0, s_base)
                start_gather(0, g_n % B, s0 + (g_n // B) * SB)

            process(rows1)
            start_writeout(1, b1, s_base)
            return 0

        lax.fori_loop(0, n_pairs, pair_body, 0)
        wait_writeout(0, B - 2, s0 + (s_per_w - SB))
        wait_writeout(1, B - 1, s0 + (s_per_w - SB))

    return k


def kernel(trg_ids, emb_table, gamma, beta):
    B, S = trg_ids.shape
    V, D = emb_table.shape
    pe = jnp.asarray(_pos_encoding_table_np(S, D))
    k = _make_sc_kernel(B, S, D, V, SB=16)
    return k(emb_table, trg_ids.astype(jnp.int32), pe, gamma, beta)


# speedup vs baseline: 1.6680x; 1.0630x over previous
import functools

import numpy as np

import jax
import jax.numpy as jnp
from jax import lax
from jax.experimental import pallas as pl
from jax.experimental.pallas import tpu as pltpu
from jax.experimental.pallas import tpu_sc as plsc

L = 16


def _pos_encoding_table_np(seq_len, d_model):
    pos = np.arange(seq_len, dtype=np.float64)[:, None]
    _2i = np.arange(0, d_model, 2, dtype=np.float64)
    angle = pos / np.power(10000.0, _2i / d_model)
    pe = np.empty((seq_len, d_model), dtype=np.float32)
    pe[:, 0::2] = np.sin(angle.astype(np.float32))
    pe[:, 1::2] = np.cos(angle.astype(np.float32))
    return pe


def _rsqrt16(v):
    half = v * 0.5
    i = plsc.bitcast(v, jnp.int32)
    i = 0x5F3759DF - (i >> 1)
    r = plsc.bitcast(i, jnp.float32)
    r = r * (1.5 - half * r * r)
    r = r * (1.5 - half * r * r)
    r = r * (1.5 - half * r * r)
    return r


def _make_sc_kernel(B, S, D, V, SB):
    info = plsc.get_sparse_core_info()
    NC, NS = info.num_cores, info.num_subcores
    NW = NC * NS
    assert S % NW == 0
    s_per_w = S // NW
    assert s_per_w % SB == 0 and SB == L
    n_sb = s_per_w // SB
    n_slices = D // L
    n_ch = n_sb * B
    assert n_ch % 2 == 0 and B % 2 == 0
    n_pairs = n_ch // 2

    mesh = plsc.VectorSubcoreMesh(core_axis_name="c", subcore_axis_name="s")

    @functools.partial(
        pl.kernel,
        mesh=mesh,
        out_type=jax.ShapeDtypeStruct((B, S, D), jnp.float32),
        compiler_params=pltpu.CompilerParams(needs_layout_passes=False),
        scratch_types=[
            pltpu.VMEM((B * (S // NW),), jnp.int32),
            pltpu.VMEM((SB, D), jnp.float32),
            pltpu.VMEM((SB, D), jnp.float32),
            pltpu.VMEM((SB, D), jnp.float32),
            pltpu.VMEM((D,), jnp.float32),
            pltpu.VMEM((D,), jnp.float32),
            pltpu.SemaphoreType.DMA,
            pltpu.SemaphoreType.DMA,
            pltpu.SemaphoreType.DMA,
            pltpu.SemaphoreType.DMA,
        ],
    )
    def k(table_hbm, ids_hbm, pe_hbm, gamma_hbm, beta_hbm, out_hbm,
          ids_all, pe_v, rows0, rows1, gamma_v, beta_v,
          sg0, sg1, sw0, sw1):
        wid = lax.axis_index("s") * NC + lax.axis_index("c")
        s0 = wid * s_per_w

        pltpu.sync_copy(gamma_hbm, gamma_v)
        pltpu.sync_copy(beta_hbm, beta_v)
        for b_ in range(B):
            pltpu.sync_copy(ids_hbm.at[b_, pl.ds(s0, s_per_w)],
                            ids_all.at[pl.ds(b_ * s_per_w, s_per_w)])

        inv_d = jnp.float32(1.0 / D)
        lanes = lax.iota(jnp.int32, L)
        zf = jnp.zeros((L,), jnp.float32)
        rows = (rows0, rows1)
        sgs = (sg0, sg1)
        sws = (sw0, sw1)

        def start_gather(kk, off):
            pltpu.make_async_copy(table_hbm.at[ids_all.at[pl.ds(off, SB)]],
                                  rows[kk], sgs[kk]).start()

        def wait_gather(kk, off):
            pltpu.make_async_copy(table_hbm.at[ids_all.at[pl.ds(off, SB)]],
                                  rows[kk], sgs[kk]).wait()

        def start_writeout(kk, b, s_base):
            pltpu.make_async_copy(rows[kk], out_hbm.at[b, pl.ds(s_base, SB)],
                                  sws[kk]).start()

        def wait_writeout(kk, b, s_base):
            pltpu.make_async_copy(rows[kk], out_hbm.at[b, pl.ds(s_base, SB)],
                                  sws[kk]).wait()

        def process(rows_v):
            def t_body(t, carry):
                sum_vec, ssq_vec = carry
                a = [zf, zf, zf, zf]
                q = [zf, zf, zf, zf]
                for j in range(n_slices):
                    sl = pl.ds(j * L, L)
                    y = rows_v[t, sl] + pe_v[t, sl]
                    rows_v[t, sl] = y
                    a[j % 4] = a[j % 4] + y
                    q[j % 4] = q[j % 4] + y * y
                s = jnp.sum((a[0] + a[1]) + (a[2] + a[3]))
                ss = jnp.sum((q[0] + q[1]) + (q[2] + q[3]))
                lane = lanes == t
                return (jnp.where(lane, s, sum_vec),
                        jnp.where(lane, ss, ssq_vec))

            sum_vec, ssq_vec = lax.fori_loop(0, SB, t_body, (zf, zf))
            mean_vec = sum_vec * inv_d
            var_vec = ssq_vec * inv_d - mean_vec * mean_vec
            rinv_vec = _rsqrt16(var_vec + jnp.float32(1e-5))

            msp = [jnp.full((L,), mean_vec[t], jnp.float32) for t in range(SB)]
            rsp = [jnp.full((L,), rinv_vec[t], jnp.float32) for t in range(SB)]

            def j_body(j, _):
                sl = pl.ds(j * L, L)
                gj = gamma_v[sl]
                bj = beta_v[sl]
                for t in range(SB):
                    y = rows_v[t, sl]
                    rows_v[t, sl] = (y - msp[t]) * rsp[t] * gj + bj
                return 0

            lax.fori_loop(0, n_slices, j_body, 0)

        start_gather(0, 0)

        def pair_body(p, _):
            g0 = p * 2
            b0 = g0 % B
            b1 = b0 + 1
            sb_l = (g0 // B) * SB
            s_base = s0 + sb_l
            off0 = b0 * s_per_w + sb_l
            off1 = off0 + s_per_w

            wait_gather(0, off0)

            @pl.when(b0 == 0)
            def _():
                pltpu.sync_copy(pe_hbm.at[pl.ds(s_base, SB)], pe_v)

            @pl.when(p > 0)
            def _():
                wait_writeout(1, b1, s_base)
            start_gather(1, off1)

            process(rows0)
            start_writeout(0, b0, s_base)

            wait_gather(1, off1)

            @pl.when(p + 1 < n_pairs)
            def _():
                g_n = g0 + 2
                wait_writeout(0, b0, s_base)
                start_gather(0, (g_n % B) * s_per_w + (g_n // B) * SB)

            process(rows1)
            start_writeout(1, b1, s_base)
            return 0

        lax.fori_loop(0, n_pairs, pair_body, 0)
        wait_writeout(0, B - 2, s0 + (s_per_w - SB))
        wait_writeout(1, B - 1, s0 + (s_per_w - SB))

    return k


def kernel(trg_ids, emb_table, gamma, beta):
    B, S = trg_ids.shape
    V, D = emb_table.shape
    pe = jnp.asarray(_pos_encoding_table_np(S, D))
    k = _make_sc_kernel(B, S, D, V, SB=16)
    return k(emb_table, trg_ids.astype(jnp.int32), pe, gamma, beta)
